# baseline (speedup 1.0000x reference)
import functools

import jax
import jax.numpy as jnp
from jax import lax
from jax.experimental import pallas as pl
from jax.experimental.pallas import tpu as pltpu
from jax.experimental.pallas import tpu_sc as plsc

NC_ = 2
NS_ = 16
NW_ = NC_ * NS_
L16_ = 16

SCALE_ = 8.0

VOCAB_ = 1000000
D_ = 64
B_ = 4096
L_ = 200

FULL_COLS_ = 7812
COLS_PER_W_ = 244


def _make_restage_kernel():
    mesh = plsc.VectorSubcoreMesh(core_axis_name="c", subcore_axis_name="s")

    @functools.partial(
        pl.kernel,
        out_type=jax.ShapeDtypeStruct((FULL_COLS_, D_, 128), jnp.float32),
        mesh=mesh,
        scratch_types=[
            pltpu.VMEM((D_, 128), jnp.float32),
            pltpu.VMEM((D_, 128), jnp.float32),
            pltpu.VMEM((D_, 128), jnp.float32),
            pltpu.VMEM((D_, 128), jnp.float32),
            pltpu.VMEM((D_, 128), jnp.float32),
            pltpu.VMEM((D_, 128), jnp.float32),
            pltpu.VMEM((D_, 128), jnp.float32),
            pltpu.VMEM((D_, 128), jnp.float32),
            pltpu.SemaphoreType.DMA,
            pltpu.SemaphoreType.DMA,
            pltpu.SemaphoreType.DMA,
            pltpu.SemaphoreType.DMA,
            pltpu.SemaphoreType.DMA,
            pltpu.SemaphoreType.DMA,
            pltpu.SemaphoreType.DMA,
            pltpu.SemaphoreType.DMA,
        ],
        compiler_params=pltpu.CompilerParams(use_tc_tiling_on_sc=True,
                                             needs_layout_passes=False),
    )
    def ka1(tbl_t, out3, b0, b1, b2, b3, b4, b5, b6, b7,
            i0, i1, i2, i3, o0, o1, o2, o3):
        wid = lax.axis_index("s") * NC_ + lax.axis_index("c")
        base = wid * COLS_PER_W_
        tins = (b0, b1, b2, b3)
        touts = (b4, b5, b6, b7)
        isems = (i0, i1, i2, i3)
        osems = (o0, o1, o2, o3)

        def copy_slab(tin, tout):
            @plsc.parallel_loop(0, D_, unroll=4)
            def _(c):
                for t0 in range(8):
                    sl = pl.ds(t0 * L16_, L16_)
                    tout[c, sl] = tin[c, sl]

        def start_in(k, b):
            pltpu.async_copy(tbl_t.at[:, pl.ds((base + k) * 128, 128)],
                             tins[b], isems[b])

        def wait_in(b):
            pltpu.make_async_copy(tbl_t.at[:, pl.ds(0, 128)], tins[b],
                                  isems[b]).wait()

        def start_out(k, b):
            pltpu.async_copy(touts[b], out3.at[base + k], osems[b])

        def drain_out(b):
            pltpu.make_async_copy(touts[b], out3.at[0], osems[b]).wait()

        for b in range(4):
            start_in(b, b)
        for b in range(4):
            wait_in(b)
            copy_slab(tins[b], touts[b])
            start_out(b, b)
            start_in(b + 4, b)

        def body(k4, carry):
            for b in range(4):
                k = 4 * k4 + b
                wait_in(b)
                drain_out(b)
                copy_slab(tins[b], touts[b])
                start_out(k, b)
                start_in(k + 4, b)
            return carry

        lax.fori_loop(1, COLS_PER_W_ // 4 - 1, body, 0)

        for b in range(4):
            k = COLS_PER_W_ - 4 + b
            wait_in(b)
            drain_out(b)
            copy_slab(tins[b], touts[b])
            start_out(k, b)
        for b in range(4):
            drain_out(b)

        @pl.when(wid < 4)
        def _():
            j = 32 * COLS_PER_W_ + wid
            pltpu.sync_copy(tbl_t.at[:, pl.ds(j * 128, 128)], b0)
            pltpu.sync_copy(b0, out3.at[j])

    return ka1


def _make_transpose_kernel():
    mesh = plsc.VectorSubcoreMesh(core_axis_name="c", subcore_axis_name="s")

    @functools.partial(
        pl.kernel,
        out_type=jax.ShapeDtypeStruct((VOCAB_, D_), jnp.float32),
        mesh=mesh,
        scratch_types=[
            pltpu.VMEM((D_, 128), jnp.float32),
            pltpu.VMEM((D_, 128), jnp.float32),
            pltpu.VMEM((D_, 128), jnp.float32),
            pltpu.VMEM((D_, 128), jnp.float32),
            pltpu.VMEM((128, 65), jnp.float32),
            pltpu.VMEM((128, 65), jnp.float32),
            pltpu.VMEM((128, 65), jnp.float32),
            pltpu.VMEM((128, 65), jnp.float32),
            pltpu.VMEM((D_, D_), jnp.float32),
            pltpu.SemaphoreType.DMA,
            pltpu.SemaphoreType.DMA,
            pltpu.SemaphoreType.DMA,
            pltpu.SemaphoreType.DMA,
            pltpu.SemaphoreType.DMA,
            pltpu.SemaphoreType.DMA,
            pltpu.SemaphoreType.DMA,
            pltpu.SemaphoreType.DMA,
        ],
        compiler_params=pltpu.CompilerParams(use_tc_tiling_on_sc=False,
                                             needs_layout_passes=False),
    )
    def ka2(slabs, t_tail, out2, tin0, tin1, tin2, tin3,
            tout0, tout1, tout2, tout3, stg,
            i0, i1, i2, i3, o0, o1, o2, o3):
        wid = lax.axis_index("s") * NC_ + lax.axis_index("c")
        base = wid * COLS_PER_W_
        tins = (tin0, tin1, tin2, tin3)
        touts = (tout0, tout1, tout2, tout3)
        isems = (i0, i1, i2, i3)
        osems = (o0, o1, o2, o3)
        iot = lax.iota(jnp.int32, L16_)
        tvecs = [iot + t0 * L16_ for t0 in range(8)]

        def transpose_slab(tin, tout):
            @plsc.parallel_loop(0, D_, unroll=8)
            def _(c):
                cv = iot * 0 + c
                for t0 in range(8):
                    v = tin[c, pl.ds(t0 * L16_, L16_)]
                    plsc.store_scatter(tout, [tvecs[t0], cv], v)

        def start_in(k, b):
            pltpu.async_copy(slabs.at[base + k], tins[b], isems[b])

        def wait_in(b):
            pltpu.make_async_copy(slabs.at[0], tins[b], isems[b]).wait()

        def start_out(k, b):
            pltpu.async_copy(touts[b].at[:, pl.ds(0, D_)],
                             out2.at[pl.ds((base + k) * 128, 128)],
                             osems[b])

        def drain_out(b):
            pltpu.make_async_copy(touts[b].at[:, pl.ds(0, D_)],
                                  out2.at[pl.ds(0, 128)], osems[b]).wait()

        for b in range(4):
            start_in(b, b)
        for b in range(4):
            wait_in(b)
            transpose_slab(tins[b], touts[b])
            start_out(b, b)
            start_in(b + 4, b)

        def body(k4, carry):
            for b in range(4):
                k = 4 * k4 + b
                wait_in(b)
                drain_out(b)
                transpose_slab(tins[b], touts[b])
                start_out(k, b)
                start_in(k + 4, b)
            return carry

        lax.fori_loop(1, COLS_PER_W_ // 4 - 1, body, 0)

        for b in range(4):
            k = COLS_PER_W_ - 4 + b
            wait_in(b)
            drain_out(b)
            transpose_slab(tins[b], touts[b])
            start_out(k, b)
        for b in range(4):
            drain_out(b)

        @pl.when(wid < 4)
        def _():
            j = 32 * COLS_PER_W_ + wid
            pltpu.sync_copy(slabs.at[j], tin0)
            transpose_slab(tin0, tout0)
            pltpu.sync_copy(tout0.at[:, pl.ds(0, D_)],
                            out2.at[pl.ds(j * 128, 128)])

        @pl.when(wid == 4)
        def _():
            pltpu.sync_copy(t_tail, stg)
            pltpu.sync_copy(stg, out2.at[pl.ds(FULL_COLS_ * 128, 64)])

    return ka2


def _make_gather_kernel():
    mesh = plsc.VectorSubcoreMesh(core_axis_name="c", subcore_axis_name="s")

    @functools.partial(
        pl.kernel,
        out_type=jax.ShapeDtypeStruct((L_, 8, 32, 8, 128), jnp.float32),
        mesh=mesh,
        scratch_types=[
            pltpu.VMEM((128,), jnp.int32),
            pltpu.VMEM((128,), jnp.int32),
            pltpu.VMEM((128,), jnp.int32),
            pltpu.VMEM((128,), jnp.int32),
            pltpu.VMEM((128, D_), jnp.float32),
            pltpu.VMEM((128, D_), jnp.float32),
            pltpu.VMEM((128, D_), jnp.float32),
            pltpu.VMEM((128, D_), jnp.float32),
            pltpu.VMEM((8, 8, 129), jnp.float32),
            pltpu.VMEM((8, 8, 129), jnp.float32),
            pltpu.VMEM((8, 8, 129), jnp.float32),
            pltpu.VMEM((8, 8, 129), jnp.float32),
            pltpu.SemaphoreType.DMA,
            pltpu.SemaphoreType.DMA,
            pltpu.SemaphoreType.DMA,
            pltpu.SemaphoreType.DMA,
            pltpu.SemaphoreType.DMA,
            pltpu.SemaphoreType.DMA,
            pltpu.SemaphoreType.DMA,
            pltpu.SemaphoreType.DMA,
        ],
        compiler_params=pltpu.CompilerParams(use_tc_tiling_on_sc=False,
                                             needs_layout_passes=False),
    )
    def kb(t64, tok_lt, out5, x0, x1, x2, x3, g0, g1, g2, g3,
           s0, s1, s2, s3, gs0, gs1, gs2, gs3, os0, os1, os2, os3):
        wid = lax.axis_index("s") * NC_ + lax.axis_index("c")
        idxs = (x0, x1, x2, x3)
        gbufs = (g0, g1, g2, g3)
        sbufs = (s0, s1, s2, s3)
        gsems = (gs0, gs1, gs2, gs3)
        osems = (os0, os1, os2, os3)
        iot = lax.iota(jnp.int32, L16_)
        zero16 = iot * 0
        c8vecs = [(iot + c0 * L16_) // 8 for c0 in range(4)]
        c2vecs = [(iot + c0 * L16_) % 8 for c0 in range(4)]

        def start(l, b):
            pltpu.sync_copy(tok_lt.at[pl.ds(l * B_ + wid * 128, 128)],
                            idxs[b])
            pltpu.async_copy(t64.at[idxs[b]], gbufs[b], gsems[b])

        def wait_gather(b):
            pltpu.make_async_copy(t64.at[idxs[b]], gbufs[b],
                                  gsems[b]).wait()

        def transpose_scale(g, s):
            @plsc.parallel_loop(0, 128, unroll=8)
            def _(t):
                tv = zero16 + t
                for c0 in range(4):
                    v = g[t, pl.ds(c0 * L16_, L16_)]
                    plsc.store_scatter(s, [c8vecs[c0], c2vecs[c0], tv],
                                       v * SCALE_)

        def start_out(l, b):
            pltpu.async_copy(sbufs[b].at[:, :, pl.ds(0, 128)],
                             out5.at[l, :, wid], osems[b])

        def drain_out(b):
            pltpu.make_async_copy(sbufs[b].at[:, :, pl.ds(0, 128)],
                                  out5.at[0, :, 0], osems[b]).wait()

        for b in range(4):
            start(b, b)
        for b in range(4):
            wait_gather(b)
            transpose_scale(gbufs[b], sbufs[b])
            start_out(b, b)
            start(b + 4, b)

        def body(k4, carry):
            for b in range(4):
                l = 4 * k4 + b
                wait_gather(b)
                drain_out(b)
                transpose_scale(gbufs[b], sbufs[b])
                start_out(l, b)
                start(l + 4, b)
            return carry

        lax.fori_loop(1, L_ // 4 - 1, body, 0)

        for b in range(4):
            l = L_ - 4 + b
            wait_gather(b)
            drain_out(b)
            transpose_scale(gbufs[b], sbufs[b])
            start_out(l, b)
        for b in range(4):
            drain_out(b)

    return kb


@jax.jit
def kernel(tokens, table):
    t_tail = table[FULL_COLS_ * 128:]
    slabs = _make_restage_kernel()(table.T)
    t64 = _make_transpose_kernel()(slabs, t_tail)
    tok_lt = tokens.astype(jnp.int32).T.reshape(B_ * L_)
    out5 = _make_gather_kernel()(t64, tok_lt)
    return out5.transpose(2, 4, 0, 1, 3).reshape(B_, L_, D_)


# speedup vs baseline: 1.0961x; 1.0961x over previous
import functools

import jax
import jax.numpy as jnp
from jax import lax
from jax.experimental import pallas as pl
from jax.experimental.pallas import tpu as pltpu
from jax.experimental.pallas import tpu_sc as plsc

NC_ = 2
NS_ = 16
NW_ = NC_ * NS_
L16_ = 16

SCALE_ = 8.0

VOCAB_ = 1000000
D_ = 64
B_ = 4096
L_ = 200

FULL_COLS_ = 7812
COLS_PER_W_ = 244


def _make_restage_kernel():
    mesh = plsc.VectorSubcoreMesh(core_axis_name="c", subcore_axis_name="s")

    @functools.partial(
        pl.kernel,
        out_type=jax.ShapeDtypeStruct((FULL_COLS_, D_, 128), jnp.float32),
        mesh=mesh,
        scratch_types=[
            pltpu.VMEM((D_, 128), jnp.float32),
            pltpu.VMEM((D_, 128), jnp.float32),
            pltpu.VMEM((D_, 128), jnp.float32),
            pltpu.VMEM((D_, 128), jnp.float32),
            pltpu.VMEM((D_, 128), jnp.float32),
            pltpu.VMEM((D_, 128), jnp.float32),
            pltpu.VMEM((D_, 128), jnp.float32),
            pltpu.VMEM((D_, 128), jnp.float32),
            pltpu.SemaphoreType.DMA,
            pltpu.SemaphoreType.DMA,
            pltpu.SemaphoreType.DMA,
            pltpu.SemaphoreType.DMA,
            pltpu.SemaphoreType.DMA,
            pltpu.SemaphoreType.DMA,
            pltpu.SemaphoreType.DMA,
            pltpu.SemaphoreType.DMA,
        ],
        compiler_params=pltpu.CompilerParams(use_tc_tiling_on_sc=True,
                                             needs_layout_passes=False),
    )
    def ka1(tbl_t, out3, b0, b1, b2, b3, b4, b5, b6, b7,
            i0, i1, i2, i3, o0, o1, o2, o3):
        wid = lax.axis_index("s") * NC_ + lax.axis_index("c")
        base = wid * COLS_PER_W_
        tins = (b0, b1, b2, b3)
        touts = (b4, b5, b6, b7)
        isems = (i0, i1, i2, i3)
        osems = (o0, o1, o2, o3)

        def copy_slab(tin, tout):
            @plsc.parallel_loop(0, D_, unroll=4)
            def _(c):
                for t0 in range(8):
                    sl = pl.ds(t0 * L16_, L16_)
                    tout[c, sl] = tin[c, sl]

        def start_in(k, b):
            pltpu.async_copy(tbl_t.at[:, pl.ds((base + k) * 128, 128)],
                             tins[b], isems[b])

        def wait_in(b):
            pltpu.make_async_copy(tbl_t.at[:, pl.ds(0, 128)], tins[b],
                                  isems[b]).wait()

        def start_out(k, b):
            pltpu.async_copy(touts[b], out3.at[base + k], osems[b])

        def drain_out(b):
            pltpu.make_async_copy(touts[b], out3.at[0], osems[b]).wait()

        for b in range(4):
            start_in(b, b)
        for b in range(4):
            wait_in(b)
            copy_slab(tins[b], touts[b])
            start_out(b, b)
            start_in(b + 4, b)

        def body(k4, carry):
            for b in range(4):
                k = 4 * k4 + b
                wait_in(b)
                drain_out(b)
                copy_slab(tins[b], touts[b])
                start_out(k, b)
                start_in(k + 4, b)
            return carry

        lax.fori_loop(1, COLS_PER_W_ // 4 - 1, body, 0)

        for b in range(4):
            k = COLS_PER_W_ - 4 + b
            wait_in(b)
            drain_out(b)
            copy_slab(tins[b], touts[b])
            start_out(k, b)
        for b in range(4):
            drain_out(b)

        @pl.when(wid < 4)
        def _():
            j = 32 * COLS_PER_W_ + wid
            pltpu.sync_copy(tbl_t.at[:, pl.ds(j * 128, 128)], b0)
            pltpu.sync_copy(b0, out3.at[j])

    return ka1


def _make_transpose_kernel():
    mesh = plsc.VectorSubcoreMesh(core_axis_name="c", subcore_axis_name="s")

    @functools.partial(
        pl.kernel,
        out_type=jax.ShapeDtypeStruct((VOCAB_, D_), jnp.float32),
        mesh=mesh,
        scratch_types=[
            pltpu.VMEM((D_, 128), jnp.float32),
            pltpu.VMEM((D_, 128), jnp.float32),
            pltpu.VMEM((D_, 128), jnp.float32),
            pltpu.VMEM((D_, 128), jnp.float32),
            pltpu.VMEM((128, 65), jnp.float32),
            pltpu.VMEM((128, 65), jnp.float32),
            pltpu.VMEM((128, 65), jnp.float32),
            pltpu.VMEM((128, 65), jnp.float32),
            pltpu.VMEM((D_, D_), jnp.float32),
            pltpu.SemaphoreType.DMA,
            pltpu.SemaphoreType.DMA,
            pltpu.SemaphoreType.DMA,
            pltpu.SemaphoreType.DMA,
            pltpu.SemaphoreType.DMA,
            pltpu.SemaphoreType.DMA,
            pltpu.SemaphoreType.DMA,
            pltpu.SemaphoreType.DMA,
        ],
        compiler_params=pltpu.CompilerParams(use_tc_tiling_on_sc=False,
                                             needs_layout_passes=False),
    )
    def ka2(slabs, t_tail, out2, tin0, tin1, tin2, tin3,
            tout0, tout1, tout2, tout3, stg,
            i0, i1, i2, i3, o0, o1, o2, o3):
        wid = lax.axis_index("s") * NC_ + lax.axis_index("c")
        base = wid * COLS_PER_W_
        tins = (tin0, tin1, tin2, tin3)
        touts = (tout0, tout1, tout2, tout3)
        isems = (i0, i1, i2, i3)
        osems = (o0, o1, o2, o3)
        iot = lax.iota(jnp.int32, L16_)
        tvecs = [iot + t0 * L16_ for t0 in range(8)]

        def transpose_slab(tin, tout):
            @plsc.parallel_loop(0, D_, unroll=4)
            def _(c):
                cv = iot * 0 + c
                for t0 in range(8):
                    v = tin[c, pl.ds(t0 * L16_, L16_)]
                    plsc.store_scatter(tout, [tvecs[t0], cv], v)

        def start_in(k, b):
            pltpu.async_copy(slabs.at[base + k], tins[b], isems[b])

        def wait_in(b):
            pltpu.make_async_copy(slabs.at[0], tins[b], isems[b]).wait()

        def start_out(k, b):
            pltpu.async_copy(touts[b].at[:, pl.ds(0, D_)],
                             out2.at[pl.ds((base + k) * 128, 128)],
                             osems[b])

        def drain_out(b):
            pltpu.make_async_copy(touts[b].at[:, pl.ds(0, D_)],
                                  out2.at[pl.ds(0, 128)], osems[b]).wait()

        for b in range(4):
            start_in(b, b)
        for b in range(4):
            wait_in(b)
            transpose_slab(tins[b], touts[b])
            start_out(b, b)
            start_in(b + 4, b)

        def body(k4, carry):
            for b in range(4):
                k = 4 * k4 + b
                wait_in(b)
                drain_out(b)
                transpose_slab(tins[b], touts[b])
                start_out(k, b)
                start_in(k + 4, b)
            return carry

        lax.fori_loop(1, COLS_PER_W_ // 4 - 1, body, 0)

        for b in range(4):
            k = COLS_PER_W_ - 4 + b
            wait_in(b)
            drain_out(b)
            transpose_slab(tins[b], touts[b])
            start_out(k, b)
        for b in range(4):
            drain_out(b)

        @pl.when(wid < 4)
        def _():
            j = 32 * COLS_PER_W_ + wid
            pltpu.sync_copy(slabs.at[j], tin0)
            transpose_slab(tin0, tout0)
            pltpu.sync_copy(tout0.at[:, pl.ds(0, D_)],
                            out2.at[pl.ds(j * 128, 128)])

        @pl.when(wid == 4)
        def _():
            pltpu.sync_copy(t_tail, stg)
            pltpu.sync_copy(stg, out2.at[pl.ds(FULL_COLS_ * 128, 64)])

    return ka2


def _make_gather_kernel():
    mesh = plsc.VectorSubcoreMesh(core_axis_name="c", subcore_axis_name="s")

    @functools.partial(
        pl.kernel,
        out_type=jax.ShapeDtypeStruct((L_, 8, 32, 8, 128), jnp.float32),
        mesh=mesh,
        scratch_types=[
            pltpu.VMEM((128,), jnp.int32),
            pltpu.VMEM((128,), jnp.int32),
            pltpu.VMEM((128,), jnp.int32),
            pltpu.VMEM((128,), jnp.int32),
            pltpu.VMEM((128, D_), jnp.float32),
            pltpu.VMEM((128, D_), jnp.float32),
            pltpu.VMEM((128, D_), jnp.float32),
            pltpu.VMEM((128, D_), jnp.float32),
            pltpu.VMEM((8, 8, 129), jnp.float32),
            pltpu.VMEM((8, 8, 129), jnp.float32),
            pltpu.VMEM((8, 8, 129), jnp.float32),
            pltpu.VMEM((8, 8, 129), jnp.float32),
            pltpu.SemaphoreType.DMA,
            pltpu.SemaphoreType.DMA,
            pltpu.SemaphoreType.DMA,
            pltpu.SemaphoreType.DMA,
            pltpu.SemaphoreType.DMA,
            pltpu.SemaphoreType.DMA,
            pltpu.SemaphoreType.DMA,
            pltpu.SemaphoreType.DMA,
        ],
        compiler_params=pltpu.CompilerParams(use_tc_tiling_on_sc=False,
                                             needs_layout_passes=False),
    )
    def kb(t64, tok_lt, out5, x0, x1, x2, x3, g0, g1, g2, g3,
           s0, s1, s2, s3, gs0, gs1, gs2, gs3, os0, os1, os2, os3):
        wid = lax.axis_index("s") * NC_ + lax.axis_index("c")
        idxs = (x0, x1, x2, x3)
        gbufs = (g0, g1, g2, g3)
        sbufs = (s0, s1, s2, s3)
        gsems = (gs0, gs1, gs2, gs3)
        osems = (os0, os1, os2, os3)
        iot = lax.iota(jnp.int32, L16_)
        zero16 = iot * 0
        c8vecs = [(iot + c0 * L16_) // 8 for c0 in range(4)]
        c2vecs = [(iot + c0 * L16_) % 8 for c0 in range(4)]

        def start(l, b):
            pltpu.sync_copy(tok_lt.at[pl.ds(l * B_ + wid * 128, 128)],
                            idxs[b])
            pltpu.async_copy(t64.at[idxs[b]], gbufs[b], gsems[b])

        def wait_gather(b):
            pltpu.make_async_copy(t64.at[idxs[b]], gbufs[b],
                                  gsems[b]).wait()

        def transpose_scale(g, s):
            @plsc.parallel_loop(0, 128, unroll=4)
            def _(t):
                tv = zero16 + t
                for c0 in range(4):
                    v = g[t, pl.ds(c0 * L16_, L16_)]
                    plsc.store_scatter(s, [c8vecs[c0], c2vecs[c0], tv],
                                       v * SCALE_)

        def start_out(l, b):
            pltpu.async_copy(sbufs[b].at[:, :, pl.ds(0, 128)],
                             out5.at[l, :, wid], osems[b])

        def drain_out(b):
            pltpu.make_async_copy(sbufs[b].at[:, :, pl.ds(0, 128)],
                                  out5.at[0, :, 0], osems[b]).wait()

        for b in range(4):
            start(b, b)
        for b in range(4):
            wait_gather(b)
            transpose_scale(gbufs[b], sbufs[b])
            start_out(b, b)
            start(b + 4, b)

        def body(k4, carry):
            for b in range(4):
                l = 4 * k4 + b
                wait_gather(b)
                drain_out(b)
                transpose_scale(gbufs[b], sbufs[b])
                start_out(l, b)
                start(l + 4, b)
            return carry

        lax.fori_loop(1, L_ // 4 - 1, body, 0)

        for b in range(4):
            l = L_ - 4 + b
            wait_gather(b)
            drain_out(b)
            transpose_scale(gbufs[b], sbufs[b])
            start_out(l, b)
        for b in range(4):
            drain_out(b)

    return kb


@jax.jit
def kernel(tokens, table):
    t_tail = table[FULL_COLS_ * 128:]
    slabs = _make_restage_kernel()(table.T)
    t64 = _make_transpose_kernel()(slabs, t_tail)
    tok_lt = tokens.astype(jnp.int32).T.reshape(B_ * L_)
    out5 = _make_gather_kernel()(t64, tok_lt)
    return out5.transpose(2, 4, 0, 1, 3).reshape(B_, L_, D_)
